# baseline (speedup 1.0000x reference)
import functools

import jax
import jax.numpy as jnp
from jax import lax
from jax.experimental import pallas as pl
from jax.experimental.pallas import tpu as pltpu
from jax.experimental.pallas import tpu_sc as plsc

H = 16
E = 8
TOPK = 2
TILE = 256
NT = 24
FCH = 1024
RT = 256
QT = 512
LN_EPS = 1e-5
NW = 32


def _ln(x, g, b):
    mu = jnp.mean(x, axis=-1, keepdims=True)
    var = jnp.mean(jnp.square(x - mu), axis=-1, keepdims=True)
    return (x - mu) * lax.rsqrt(var + LN_EPS) * g + b


def _ln_qkv_body(x_ref, g_ref, b_ref, w_ref, o_ref):
    h = _ln(x_ref[...], g_ref[...], b_ref[...])
    o_ref[...] = jnp.dot(h.astype(jnp.bfloat16), w_ref[...],
                         preferred_element_type=jnp.float32)


def _attn_body(q_ref, k_ref, v_ref, o_ref, *, scale, hd):
    outs = []
    for t in range(2):
        q = q_ref[:, t * hd:(t + 1) * hd].astype(jnp.bfloat16)
        k = k_ref[:, t * hd:(t + 1) * hd].astype(jnp.bfloat16)
        v = v_ref[:, t * hd:(t + 1) * hd].astype(jnp.bfloat16)
        s = lax.dot_general(q, k, (((1,), (1,)), ((), ())),
                            preferred_element_type=jnp.float32) * scale
        m = jnp.max(s, axis=-1, keepdims=True)
        p = jnp.exp(s - m)
        p = (p / jnp.sum(p, axis=-1, keepdims=True)).astype(jnp.bfloat16)
        outs.append(jnp.dot(p, v, preferred_element_type=jnp.float32))
    o_ref[...] = jnp.concatenate(outs, axis=1)


def _proj_body(a_ref, wp_ref, bp_ref, x_ref, g_ref, b_ref, f0_ref, wg_ref,
               xres_ref, hs_ref, lg_ref):
    xr = x_ref[...] + jnp.dot(a_ref[...].astype(jnp.bfloat16), wp_ref[...],
                              preferred_element_type=jnp.float32) + bp_ref[...]
    xres_ref[...] = xr
    hs = _ln(xr, g_ref[...], b_ref[...])
    hs_ref[...] = hs
    lg_ref[...] = jnp.dot(hs + f0_ref[...], wg_ref[...],
                          preferred_element_type=jnp.float32)


def _route_body(lg_ref, dst_ref, wts_ref, texp_ref, pos_ref, oh_ref, *, T, nt):
    lg = lg_ref[...]
    m = jnp.max(lg, axis=-1, keepdims=True)
    p = jnp.exp(lg - m)
    rw = p / jnp.sum(p, axis=-1, keepdims=True)
    elane = lax.broadcasted_iota(jnp.int32, (T, E), 1)
    m1 = jnp.max(rw, axis=-1, keepdims=True)
    s1 = jnp.min(jnp.where(rw == m1, elane, E), axis=-1, keepdims=True)
    rw2 = jnp.where(elane == s1, -1.0, rw)
    m2 = jnp.max(rw2, axis=-1, keepdims=True)
    s2 = jnp.min(jnp.where(rw2 == m2, elane, E), axis=-1, keepdims=True)
    w1 = m1 / (m1 + m2)
    w2 = m2 / (m1 + m2)
    oh_ref[...] = ((elane == s1) | (elane == s2)).astype(jnp.float32)

    CH = 128
    r_i = lax.broadcasted_iota(jnp.int32, (CH, CH), 0)
    c_i = lax.broadcasted_iota(jnp.int32, (CH, CH), 1)
    tril = (r_i > c_i).astype(jnp.float32)

    def chunk_step(c, carry):
        chunk = oh_ref[pl.ds(c * CH, CH), :]
        within = jnp.dot(tril, chunk, preferred_element_type=jnp.float32)
        pos_ref[pl.ds(c * CH, CH), :] = within + carry
        return carry + jnp.sum(chunk, axis=0, keepdims=True)

    counts = lax.fori_loop(0, T // CH, chunk_step,
                           jnp.zeros((1, E), jnp.float32))

    tp = (counts.astype(jnp.int32) + (TILE - 1)) >> 8
    tp_f = tp.astype(jnp.float32)
    a8 = lax.broadcasted_iota(jnp.int32, (E, E), 0)
    b8 = lax.broadcasted_iota(jnp.int32, (E, E), 1)
    mlt = (a8 < b8).astype(jnp.float32)
    tb_f = jnp.dot(tp_f, mlt, preferred_element_type=jnp.float32)
    ends_f = tb_f + tp_f
    eye = (a8 == b8).astype(jnp.float32)
    ends_col = lax.dot_general(eye, ends_f, (((1,), (1,)), ((), ())),
                               preferred_element_type=jnp.float32)
    ti = lax.broadcasted_iota(jnp.int32, (E, 32), 1).astype(jnp.float32)
    texp = jnp.sum((ends_col <= ti).astype(jnp.float32), axis=0, keepdims=True)
    texp_ref[...] = jnp.minimum(texp, E - 1).astype(jnp.int32)[:, :nt]

    pos = pos_ref[...]
    val = TILE * tb_f + pos
    d1 = jnp.sum(jnp.where(elane == s1, val, 0.0), axis=-1, keepdims=True)
    d2 = jnp.sum(jnp.where(elane == s2, val, 0.0), axis=-1, keepdims=True)
    dst_ref[...] = jnp.concatenate([d1, d2], axis=1).astype(jnp.int32)
    wts_ref[...] = jnp.concatenate([w1, w2], axis=1)


def _moe_body(texp_ref, buf_ref, wg_ref, wu_ref, wd_ref, o_ref, *, nfch):
    j = pl.program_id(1)
    x = buf_ref[...].astype(jnp.bfloat16)
    g = jnp.dot(x, wg_ref[0], preferred_element_type=jnp.float32)
    u = jnp.dot(x, wu_ref[0], preferred_element_type=jnp.float32)
    pr = (0.5 * g * (1.0 + lax.erf(g * (2.0 ** -0.5)))) * u
    d = jnp.dot(pr.astype(jnp.bfloat16), wd_ref[0],
                preferred_element_type=jnp.float32)

    @pl.when(j == 0)
    def _():
        o_ref[...] = d

    @pl.when(j > 0)
    def _():
        o_ref[...] += d


def _final_body(xr_ref, y0_ref, y1_ref, w_ref, o_ref):
    w = w_ref[...]
    o_ref[...] = (xr_ref[...] + w[:, 0:1] * y0_ref[...]
                  + w[:, 1:2] * y1_ref[...])


def _sc_dispatch(hs, idx3, CAP):
    N, C = hs.shape
    RPW = N // NW
    mesh = plsc.VectorSubcoreMesh(core_axis_name="c", subcore_axis_name="s")

    @functools.partial(
        pl.kernel,
        out_type=jax.ShapeDtypeStruct((CAP, C), jnp.float32),
        mesh=mesh,
        scratch_types=[
            pltpu.VMEM((TOPK, RPW), jnp.int32),
            pltpu.VMEM((RPW, C), jnp.float32),
            pltpu.SemaphoreType.DMA,
        ],
    )
    def _dispatch(hs_hbm, idx_hbm, buf_hbm, idx_v, rows_v, sem):
        wid = lax.axis_index("s") * 2 + lax.axis_index("c")
        pltpu.sync_copy(idx_hbm.at[wid], idx_v)
        pltpu.sync_copy(hs_hbm.at[pl.ds(wid * RPW, RPW)], rows_v)
        pltpu.async_copy(rows_v, buf_hbm.at[idx_v.at[0]], sem).wait()
        pltpu.async_copy(rows_v, buf_hbm.at[idx_v.at[1]], sem).wait()

    return _dispatch(hs, idx3)


def _sc_combine(obuf, idx3, N):
    C = obuf.shape[1]
    RPW = N // NW
    mesh = plsc.VectorSubcoreMesh(core_axis_name="c", subcore_axis_name="s")

    @functools.partial(
        pl.kernel,
        out_type=[
            jax.ShapeDtypeStruct((N, C), jnp.float32),
            jax.ShapeDtypeStruct((N, C), jnp.float32),
        ],
        mesh=mesh,
        scratch_types=[
            pltpu.VMEM((TOPK, RPW), jnp.int32),
            pltpu.VMEM((RPW, C), jnp.float32),
            pltpu.SemaphoreType.DMA,
        ],
    )
    def _combine(obuf_hbm, idx_hbm, y0_hbm, y1_hbm, idx_v, rows_v, sem):
        wid = lax.axis_index("s") * 2 + lax.axis_index("c")
        pltpu.sync_copy(idx_hbm.at[wid], idx_v)
        pltpu.async_copy(obuf_hbm.at[idx_v.at[0]], rows_v, sem).wait()
        pltpu.sync_copy(rows_v, y0_hbm.at[pl.ds(wid * RPW, RPW)])
        pltpu.async_copy(obuf_hbm.at[idx_v.at[1]], rows_v, sem).wait()
        pltpu.sync_copy(rows_v, y1_hbm.at[pl.ds(wid * RPW, RPW)])

    return _combine(obuf, idx3)


def kernel(x, g1, b1, Wq, Wkv, Wproj, bproj, g2, b2, Wg, feat0, w_gate,
           w_up, w_down, modality_length):
    Bx, N, C = x.shape
    FF = w_gate.shape[2]
    hd = C // H
    scale = hd ** (-0.5)
    CAP = NT * TILE
    RPW = N // NW

    x2 = x.reshape(N, C)
    Wqkv = jnp.concatenate([Wq, Wkv], axis=1).astype(jnp.bfloat16)
    Wproj_b = Wproj.astype(jnp.bfloat16)
    wg_b = w_gate.astype(jnp.bfloat16)
    wu_b = w_up.astype(jnp.bfloat16)
    wd_b = w_down.astype(jnp.bfloat16)
    g1r, b1r = g1.reshape(1, C), b1.reshape(1, C)
    g2r, b2r = g2.reshape(1, C), b2.reshape(1, C)
    bpr, f0r = bproj.reshape(1, C), feat0.reshape(1, C)

    qkv = pl.pallas_call(
        _ln_qkv_body,
        grid=(N // RT,),
        in_specs=[
            pl.BlockSpec((RT, C), lambda i: (i, 0)),
            pl.BlockSpec((1, C), lambda i: (0, 0)),
            pl.BlockSpec((1, C), lambda i: (0, 0)),
            pl.BlockSpec((C, 3 * C), lambda i: (0, 0)),
        ],
        out_specs=pl.BlockSpec((RT, 3 * C), lambda i: (i, 0)),
        out_shape=jax.ShapeDtypeStruct((N, 3 * C), jnp.float32),
    )(x2, g1r, b1r, Wqkv)

    HP = H // 2
    CB = C // 128
    a = pl.pallas_call(
        functools.partial(_attn_body, scale=scale, hd=hd),
        grid=(HP, N // QT),
        in_specs=[
            pl.BlockSpec((QT, 2 * hd), lambda h, i: (i, h)),
            pl.BlockSpec((N, 2 * hd), lambda h, i: (0, CB + h)),
            pl.BlockSpec((N, 2 * hd), lambda h, i: (0, 2 * CB + h)),
        ],
        out_specs=pl.BlockSpec((QT, 2 * hd), lambda h, i: (i, h)),
        out_shape=jax.ShapeDtypeStruct((N, C), jnp.float32),
    )(qkv, qkv, qkv)

    xres, hs, logits = pl.pallas_call(
        _proj_body,
        grid=(N // RT,),
        in_specs=[
            pl.BlockSpec((RT, C), lambda i: (i, 0)),
            pl.BlockSpec((C, C), lambda i: (0, 0)),
            pl.BlockSpec((1, C), lambda i: (0, 0)),
            pl.BlockSpec((RT, C), lambda i: (i, 0)),
            pl.BlockSpec((1, C), lambda i: (0, 0)),
            pl.BlockSpec((1, C), lambda i: (0, 0)),
            pl.BlockSpec((1, C), lambda i: (0, 0)),
            pl.BlockSpec((C, E), lambda i: (0, 0)),
        ],
        out_specs=[
            pl.BlockSpec((RT, C), lambda i: (i, 0)),
            pl.BlockSpec((RT, C), lambda i: (i, 0)),
            pl.BlockSpec((RT, E), lambda i: (i, 0)),
        ],
        out_shape=[
            jax.ShapeDtypeStruct((N, C), jnp.float32),
            jax.ShapeDtypeStruct((N, C), jnp.float32),
            jax.ShapeDtypeStruct((N, E), jnp.float32),
        ],
    )(a, Wproj_b, bpr, x2, g2r, b2r, f0r, Wg)

    dst, wts, texp = pl.pallas_call(
        functools.partial(_route_body, T=N, nt=NT),
        in_specs=[pl.BlockSpec((N, E), lambda: (0, 0))],
        out_specs=[
            pl.BlockSpec((N, TOPK), lambda: (0, 0)),
            pl.BlockSpec((N, TOPK), lambda: (0, 0)),
            pl.BlockSpec((1, NT), lambda: (0, 0)),
        ],
        out_shape=[
            jax.ShapeDtypeStruct((N, TOPK), jnp.int32),
            jax.ShapeDtypeStruct((N, TOPK), jnp.float32),
            jax.ShapeDtypeStruct((1, NT), jnp.int32),
        ],
        scratch_shapes=[pltpu.VMEM((N, E), jnp.float32),
                        pltpu.VMEM((N, E), jnp.float32)],
    )(logits)

    texp_flat = texp.reshape(NT)
    idx3 = dst.reshape(NW, RPW, TOPK).transpose(0, 2, 1)

    buf = _sc_dispatch(hs, idx3, CAP)

    nfch = FF // FCH
    grid_spec = pltpu.PrefetchScalarGridSpec(
        num_scalar_prefetch=1,
        grid=(NT, nfch),
        in_specs=[
            pl.BlockSpec((TILE, C), lambda i, j, te: (i, 0)),
            pl.BlockSpec((1, C, FCH), lambda i, j, te: (te[i], 0, j)),
            pl.BlockSpec((1, C, FCH), lambda i, j, te: (te[i], 0, j)),
            pl.BlockSpec((1, FCH, C), lambda i, j, te: (te[i], j, 0)),
        ],
        out_specs=pl.BlockSpec((TILE, C), lambda i, j, te: (i, 0)),
    )
    obuf = pl.pallas_call(
        functools.partial(_moe_body, nfch=nfch),
        grid_spec=grid_spec,
        out_shape=jax.ShapeDtypeStruct((CAP, C), jnp.float32),
    )(texp_flat, buf, wg_b, wu_b, wd_b)

    y0, y1 = _sc_combine(obuf, idx3, N)

    out = pl.pallas_call(
        _final_body,
        grid=(N // RT,),
        in_specs=[
            pl.BlockSpec((RT, C), lambda i: (i, 0)),
            pl.BlockSpec((RT, C), lambda i: (i, 0)),
            pl.BlockSpec((RT, C), lambda i: (i, 0)),
            pl.BlockSpec((RT, TOPK), lambda i: (i, 0)),
        ],
        out_specs=pl.BlockSpec((RT, C), lambda i: (i, 0)),
        out_shape=jax.ShapeDtypeStruct((N, C), jnp.float32),
    )(xres, y0, y1, wts)

    return out.reshape(Bx, N, C)


# speedup vs baseline: 1.2534x; 1.2534x over previous
import functools

import jax
import jax.numpy as jnp
from jax import lax
from jax.experimental import pallas as pl
from jax.experimental.pallas import tpu as pltpu
from jax.experimental.pallas import tpu_sc as plsc

H = 16
E = 8
TOPK = 2
TILE = 256
NT = 24
FCH = 1024
RT = 256
QT = 512
LN_EPS = 1e-5
NW = 32


def _ln(x, g, b):
    mu = jnp.mean(x, axis=-1, keepdims=True)
    var = jnp.mean(jnp.square(x - mu), axis=-1, keepdims=True)
    return (x - mu) * lax.rsqrt(var + LN_EPS) * g + b


def _ln_qkv_body(x_ref, g_ref, b_ref, w_ref, o_ref):
    h = _ln(x_ref[...], g_ref[...], b_ref[...])
    o_ref[...] = jnp.dot(h.astype(jnp.bfloat16),
                         w_ref[...].astype(jnp.bfloat16),
                         preferred_element_type=jnp.float32)


def _attn_body(q_ref, k_ref, v_ref, o_ref, *, scale, hd):
    outs = []
    for t in range(2):
        q = q_ref[:, t * hd:(t + 1) * hd].astype(jnp.bfloat16)
        k = k_ref[:, t * hd:(t + 1) * hd].astype(jnp.bfloat16)
        v = v_ref[:, t * hd:(t + 1) * hd].astype(jnp.bfloat16)
        s = lax.dot_general(q, k, (((1,), (1,)), ((), ())),
                            preferred_element_type=jnp.float32) * scale
        m = jnp.max(s, axis=-1, keepdims=True)
        p = jnp.exp(s - m)
        p = (p / jnp.sum(p, axis=-1, keepdims=True)).astype(jnp.bfloat16)
        outs.append(jnp.dot(p, v, preferred_element_type=jnp.float32))
    o_ref[...] = jnp.concatenate(outs, axis=1)


def _proj_body(a_ref, wp_ref, bp_ref, x_ref, g_ref, b_ref, f0_ref, wg_ref,
               xres_ref, hs_ref, lg_ref):
    xr = x_ref[...] + jnp.dot(a_ref[...].astype(jnp.bfloat16),
                              wp_ref[...].astype(jnp.bfloat16),
                              preferred_element_type=jnp.float32) + bp_ref[...]
    xres_ref[...] = xr
    hs = _ln(xr, g_ref[...], b_ref[...])
    hs_ref[...] = hs
    lg_ref[...] = jnp.dot(hs + f0_ref[...], wg_ref[...],
                          preferred_element_type=jnp.float32)


def _route_body(lg_ref, dst_ref, wts_ref, texp_ref, pos_ref, oh_ref, *, T, nt):
    lg = lg_ref[...]
    m = jnp.max(lg, axis=-1, keepdims=True)
    p = jnp.exp(lg - m)
    rw = p / jnp.sum(p, axis=-1, keepdims=True)
    elane = lax.broadcasted_iota(jnp.int32, (T, E), 1)
    m1 = jnp.max(rw, axis=-1, keepdims=True)
    s1 = jnp.min(jnp.where(rw == m1, elane, E), axis=-1, keepdims=True)
    rw2 = jnp.where(elane == s1, -1.0, rw)
    m2 = jnp.max(rw2, axis=-1, keepdims=True)
    s2 = jnp.min(jnp.where(rw2 == m2, elane, E), axis=-1, keepdims=True)
    w1 = m1 / (m1 + m2)
    w2 = m2 / (m1 + m2)
    oh_ref[...] = ((elane == s1) | (elane == s2)).astype(jnp.float32)

    CH = 128
    r_i = lax.broadcasted_iota(jnp.int32, (CH, CH), 0)
    c_i = lax.broadcasted_iota(jnp.int32, (CH, CH), 1)
    tril = (r_i > c_i).astype(jnp.float32)

    def chunk_step(c, carry):
        chunk = oh_ref[pl.ds(c * CH, CH), :]
        within = jnp.dot(tril, chunk, preferred_element_type=jnp.float32)
        pos_ref[pl.ds(c * CH, CH), :] = within + carry
        return carry + jnp.sum(chunk, axis=0, keepdims=True)

    counts = lax.fori_loop(0, T // CH, chunk_step,
                           jnp.zeros((1, E), jnp.float32))

    tp = (counts.astype(jnp.int32) + (TILE - 1)) >> 8
    tp_f = tp.astype(jnp.float32)
    a8 = lax.broadcasted_iota(jnp.int32, (E, E), 0)
    b8 = lax.broadcasted_iota(jnp.int32, (E, E), 1)
    mlt = (a8 < b8).astype(jnp.float32)
    tb_f = jnp.dot(tp_f, mlt, preferred_element_type=jnp.float32)
    ends_f = tb_f + tp_f
    eye = (a8 == b8).astype(jnp.float32)
    ends_col = lax.dot_general(eye, ends_f, (((1,), (1,)), ((), ())),
                               preferred_element_type=jnp.float32)
    ti = lax.broadcasted_iota(jnp.int32, (E, 32), 1).astype(jnp.float32)
    texp = jnp.sum((ends_col <= ti).astype(jnp.float32), axis=0, keepdims=True)
    texp_ref[...] = jnp.minimum(texp, E - 1).astype(jnp.int32)[:, :nt]

    pos = pos_ref[...]
    val = TILE * tb_f + pos
    d1 = jnp.sum(jnp.where(elane == s1, val, 0.0), axis=-1, keepdims=True)
    d2 = jnp.sum(jnp.where(elane == s2, val, 0.0), axis=-1, keepdims=True)
    dst_ref[...] = jnp.concatenate([d1, d2], axis=1).astype(jnp.int32)
    wts_ref[...] = jnp.concatenate([w1, w2], axis=1)


def _moe_body(texp_ref, buf_ref, wg_ref, wu_ref, wd_ref, o_ref, acc_ref, *,
              nfch, tile):
    j = pl.program_id(0)
    i = pl.program_id(1)
    x = buf_ref[...].astype(jnp.bfloat16)
    g = jnp.dot(x, wg_ref[0].astype(jnp.bfloat16),
                preferred_element_type=jnp.float32)
    u = jnp.dot(x, wu_ref[0].astype(jnp.bfloat16),
                preferred_element_type=jnp.float32)
    pr = (0.5 * g * (1.0 + lax.erf(g * (2.0 ** -0.5)))) * u
    d = jnp.dot(pr.astype(jnp.bfloat16), wd_ref[0].astype(jnp.bfloat16),
                preferred_element_type=jnp.float32)

    @pl.when(j == 0)
    def _():
        acc_ref[pl.ds(i * tile, tile), :] = d

    @pl.when((j > 0) & (j < nfch - 1))
    def _():
        acc_ref[pl.ds(i * tile, tile), :] += d

    @pl.when(j == nfch - 1)
    def _():
        o_ref[...] = acc_ref[pl.ds(i * tile, tile), :] + d


def _final_body(xr_ref, y0_ref, y1_ref, w_ref, o_ref):
    w = w_ref[...]
    o_ref[...] = (xr_ref[...] + w[:, 0:1] * y0_ref[...]
                  + w[:, 1:2] * y1_ref[...])


def _sc_dispatch(hs, idx3, CAP):
    N, C = hs.shape
    RPW = N // NW
    mesh = plsc.VectorSubcoreMesh(core_axis_name="c", subcore_axis_name="s")

    @functools.partial(
        pl.kernel,
        out_type=jax.ShapeDtypeStruct((CAP, C), jnp.float32),
        mesh=mesh,
        scratch_types=[
            pltpu.VMEM((TOPK, RPW), jnp.int32),
            pltpu.VMEM((RPW, C), jnp.float32),
            pltpu.SemaphoreType.DMA,
        ],
    )
    def _dispatch(hs_hbm, idx_hbm, buf_hbm, idx_v, rows_v, sem):
        wid = lax.axis_index("s") * 2 + lax.axis_index("c")
        pltpu.sync_copy(idx_hbm.at[wid], idx_v)
        pltpu.sync_copy(hs_hbm.at[pl.ds(wid * RPW, RPW)], rows_v)
        pltpu.async_copy(rows_v, buf_hbm.at[idx_v.at[0]], sem).wait()
        pltpu.async_copy(rows_v, buf_hbm.at[idx_v.at[1]], sem).wait()

    return _dispatch(hs, idx3)


def _sc_combine(obuf, idx3, N):
    C = obuf.shape[1]
    RPW = N // NW
    mesh = plsc.VectorSubcoreMesh(core_axis_name="c", subcore_axis_name="s")

    @functools.partial(
        pl.kernel,
        out_type=[
            jax.ShapeDtypeStruct((N, C), jnp.float32),
            jax.ShapeDtypeStruct((N, C), jnp.float32),
        ],
        mesh=mesh,
        scratch_types=[
            pltpu.VMEM((TOPK, RPW), jnp.int32),
            pltpu.VMEM((RPW, C), jnp.float32),
            pltpu.SemaphoreType.DMA,
        ],
    )
    def _combine(obuf_hbm, idx_hbm, y0_hbm, y1_hbm, idx_v, rows_v, sem):
        wid = lax.axis_index("s") * 2 + lax.axis_index("c")
        pltpu.sync_copy(idx_hbm.at[wid], idx_v)
        pltpu.async_copy(obuf_hbm.at[idx_v.at[0]], rows_v, sem).wait()
        pltpu.sync_copy(rows_v, y0_hbm.at[pl.ds(wid * RPW, RPW)])
        pltpu.async_copy(obuf_hbm.at[idx_v.at[1]], rows_v, sem).wait()
        pltpu.sync_copy(rows_v, y1_hbm.at[pl.ds(wid * RPW, RPW)])

    return _combine(obuf, idx3)


def kernel(x, g1, b1, Wq, Wkv, Wproj, bproj, g2, b2, Wg, feat0, w_gate,
           w_up, w_down, modality_length):
    Bx, N, C = x.shape
    FF = w_gate.shape[2]
    hd = C // H
    scale = hd ** (-0.5)
    CAP = NT * TILE
    RPW = N // NW

    x2 = x.reshape(N, C)
    Wqkv = jnp.concatenate([Wq, Wkv], axis=1)
    g1r, b1r = g1.reshape(1, C), b1.reshape(1, C)
    g2r, b2r = g2.reshape(1, C), b2.reshape(1, C)
    bpr, f0r = bproj.reshape(1, C), feat0.reshape(1, C)

    qkv = pl.pallas_call(
        _ln_qkv_body,
        grid=(N // RT,),
        in_specs=[
            pl.BlockSpec((RT, C), lambda i: (i, 0)),
            pl.BlockSpec((1, C), lambda i: (0, 0)),
            pl.BlockSpec((1, C), lambda i: (0, 0)),
            pl.BlockSpec((C, 3 * C), lambda i: (0, 0)),
        ],
        out_specs=pl.BlockSpec((RT, 3 * C), lambda i: (i, 0)),
        out_shape=jax.ShapeDtypeStruct((N, 3 * C), jnp.float32),
    )(x2, g1r, b1r, Wqkv)

    HP = H // 2
    CB = C // 128
    a = pl.pallas_call(
        functools.partial(_attn_body, scale=scale, hd=hd),
        grid=(HP, N // QT),
        in_specs=[
            pl.BlockSpec((QT, 2 * hd), lambda h, i: (i, h)),
            pl.BlockSpec((N, 2 * hd), lambda h, i: (0, CB + h)),
            pl.BlockSpec((N, 2 * hd), lambda h, i: (0, 2 * CB + h)),
        ],
        out_specs=pl.BlockSpec((QT, 2 * hd), lambda h, i: (i, h)),
        out_shape=jax.ShapeDtypeStruct((N, C), jnp.float32),
    )(qkv, qkv, qkv)

    xres, hs, logits = pl.pallas_call(
        _proj_body,
        grid=(N // RT,),
        in_specs=[
            pl.BlockSpec((RT, C), lambda i: (i, 0)),
            pl.BlockSpec((C, C), lambda i: (0, 0)),
            pl.BlockSpec((1, C), lambda i: (0, 0)),
            pl.BlockSpec((RT, C), lambda i: (i, 0)),
            pl.BlockSpec((1, C), lambda i: (0, 0)),
            pl.BlockSpec((1, C), lambda i: (0, 0)),
            pl.BlockSpec((1, C), lambda i: (0, 0)),
            pl.BlockSpec((C, E), lambda i: (0, 0)),
        ],
        out_specs=[
            pl.BlockSpec((RT, C), lambda i: (i, 0)),
            pl.BlockSpec((RT, C), lambda i: (i, 0)),
            pl.BlockSpec((RT, E), lambda i: (i, 0)),
        ],
        out_shape=[
            jax.ShapeDtypeStruct((N, C), jnp.float32),
            jax.ShapeDtypeStruct((N, C), jnp.float32),
            jax.ShapeDtypeStruct((N, E), jnp.float32),
        ],
    )(a, Wproj, bpr, x2, g2r, b2r, f0r, Wg)

    dst, wts, texp = pl.pallas_call(
        functools.partial(_route_body, T=N, nt=NT),
        in_specs=[pl.BlockSpec((N, E), lambda: (0, 0))],
        out_specs=[
            pl.BlockSpec((N, TOPK), lambda: (0, 0)),
            pl.BlockSpec((N, TOPK), lambda: (0, 0)),
            pl.BlockSpec((1, NT), lambda: (0, 0)),
        ],
        out_shape=[
            jax.ShapeDtypeStruct((N, TOPK), jnp.int32),
            jax.ShapeDtypeStruct((N, TOPK), jnp.float32),
            jax.ShapeDtypeStruct((1, NT), jnp.int32),
        ],
        scratch_shapes=[pltpu.VMEM((N, E), jnp.float32),
                        pltpu.VMEM((N, E), jnp.float32)],
    )(logits)

    texp_flat = texp.reshape(NT)
    idx3 = dst.reshape(NW, RPW, TOPK).transpose(0, 2, 1)

    buf = _sc_dispatch(hs, idx3, CAP)

    nfch = FF // FCH
    grid_spec = pltpu.PrefetchScalarGridSpec(
        num_scalar_prefetch=1,
        grid=(nfch, NT),
        in_specs=[
            pl.BlockSpec((TILE, C), lambda j, i, te: (i, 0)),
            pl.BlockSpec((1, C, FCH), lambda j, i, te: (te[i], 0, j)),
            pl.BlockSpec((1, C, FCH), lambda j, i, te: (te[i], 0, j)),
            pl.BlockSpec((1, FCH, C), lambda j, i, te: (te[i], j, 0)),
        ],
        out_specs=pl.BlockSpec(
            (TILE, C),
            lambda j, i, te: (jnp.where(j == nfch - 1, i, 0), 0)),
        scratch_shapes=[pltpu.VMEM((NT * TILE, C), jnp.float32)],
    )
    obuf = pl.pallas_call(
        functools.partial(_moe_body, nfch=nfch, tile=TILE),
        grid_spec=grid_spec,
        out_shape=jax.ShapeDtypeStruct((CAP, C), jnp.float32),
    )(texp_flat, buf, w_gate, w_up, w_down)

    y0, y1 = _sc_combine(obuf, idx3, N)

    out = pl.pallas_call(
        _final_body,
        grid=(N // RT,),
        in_specs=[
            pl.BlockSpec((RT, C), lambda i: (i, 0)),
            pl.BlockSpec((RT, C), lambda i: (i, 0)),
            pl.BlockSpec((RT, C), lambda i: (i, 0)),
            pl.BlockSpec((RT, TOPK), lambda i: (i, 0)),
        ],
        out_specs=pl.BlockSpec((RT, C), lambda i: (i, 0)),
        out_shape=jax.ShapeDtypeStruct((N, C), jnp.float32),
    )(xres, y0, y1, wts)

    return out.reshape(Bx, N, C)


# speedup vs baseline: 1.4345x; 1.1445x over previous
import functools

import jax
import jax.numpy as jnp
from jax import lax
from jax.experimental import pallas as pl
from jax.experimental.pallas import tpu as pltpu
from jax.experimental.pallas import tpu_sc as plsc

H = 16
E = 8
TOPK = 2
TILE = 256
NT = 24
FCH = 1024
RT = 256
QT = 512
LN_EPS = 1e-5
NW = 32


def _ln(x, g, b):
    mu = jnp.mean(x, axis=-1, keepdims=True)
    var = jnp.mean(jnp.square(x - mu), axis=-1, keepdims=True)
    return (x - mu) * lax.rsqrt(var + LN_EPS) * g + b


def _ln_qkv_body(x_ref, g_ref, b_ref, w_ref, o_ref):
    h = _ln(x_ref[...], g_ref[...], b_ref[...])
    o_ref[...] = jnp.dot(h.astype(jnp.bfloat16),
                         w_ref[...].astype(jnp.bfloat16),
                         preferred_element_type=jnp.float32)


def _attn_body(q_ref, k_ref, v_ref, o_ref, *, scale, hd):
    outs = []
    for t in range(2):
        q = q_ref[:, t * hd:(t + 1) * hd].astype(jnp.bfloat16)
        k = k_ref[:, t * hd:(t + 1) * hd].astype(jnp.bfloat16)
        v = v_ref[:, t * hd:(t + 1) * hd].astype(jnp.bfloat16)
        s = lax.dot_general(q, k, (((1,), (1,)), ((), ())),
                            preferred_element_type=jnp.float32) * scale
        p = jnp.exp(s)
        r = 1.0 / jnp.sum(p, axis=-1, keepdims=True)
        pv = jnp.dot(p.astype(jnp.bfloat16), v,
                     preferred_element_type=jnp.float32)
        outs.append(pv * r)
    o_ref[...] = jnp.concatenate(outs, axis=1)


def _proj_body(a_ref, wp_ref, bp_ref, x_ref, g_ref, b_ref, f0_ref, wg_ref,
               xres_ref, hs_ref, lg_ref):
    xr = x_ref[...] + jnp.dot(a_ref[...].astype(jnp.bfloat16),
                              wp_ref[...].astype(jnp.bfloat16),
                              preferred_element_type=jnp.float32) + bp_ref[...]
    xres_ref[...] = xr
    hs = _ln(xr, g_ref[...], b_ref[...])
    hs_ref[...] = hs
    lg_ref[...] = jnp.dot(hs + f0_ref[...], wg_ref[...],
                          preferred_element_type=jnp.float32)


def _route_body(lg_ref, dst_ref, wts_ref, texp_ref, pos_ref, oh_ref, *, T, nt):
    lg = lg_ref[...]
    m = jnp.max(lg, axis=-1, keepdims=True)
    p = jnp.exp(lg - m)
    rw = p / jnp.sum(p, axis=-1, keepdims=True)
    elane = lax.broadcasted_iota(jnp.int32, (T, E), 1)
    m1 = jnp.max(rw, axis=-1, keepdims=True)
    s1 = jnp.min(jnp.where(rw == m1, elane, E), axis=-1, keepdims=True)
    rw2 = jnp.where(elane == s1, -1.0, rw)
    m2 = jnp.max(rw2, axis=-1, keepdims=True)
    s2 = jnp.min(jnp.where(rw2 == m2, elane, E), axis=-1, keepdims=True)
    w1 = m1 / (m1 + m2)
    w2 = m2 / (m1 + m2)
    oh_ref[...] = ((elane == s1) | (elane == s2)).astype(jnp.float32)

    CH = 128
    r_i = lax.broadcasted_iota(jnp.int32, (CH, CH), 0)
    c_i = lax.broadcasted_iota(jnp.int32, (CH, CH), 1)
    tril = (r_i > c_i).astype(jnp.float32)

    def chunk_step(c, carry):
        chunk = oh_ref[pl.ds(c * CH, CH), :]
        within = jnp.dot(tril, chunk, preferred_element_type=jnp.float32)
        pos_ref[pl.ds(c * CH, CH), :] = within + carry
        return carry + jnp.sum(chunk, axis=0, keepdims=True)

    counts = lax.fori_loop(0, T // CH, chunk_step,
                           jnp.zeros((1, E), jnp.float32))

    tp = (counts.astype(jnp.int32) + (TILE - 1)) >> 8
    tp_f = tp.astype(jnp.float32)
    a8 = lax.broadcasted_iota(jnp.int32, (E, E), 0)
    b8 = lax.broadcasted_iota(jnp.int32, (E, E), 1)
    mlt = (a8 < b8).astype(jnp.float32)
    tb_f = jnp.dot(tp_f, mlt, preferred_element_type=jnp.float32)
    ends_f = tb_f + tp_f
    eye = (a8 == b8).astype(jnp.float32)
    ends_col = lax.dot_general(eye, ends_f, (((1,), (1,)), ((), ())),
                               preferred_element_type=jnp.float32)
    ti = lax.broadcasted_iota(jnp.int32, (E, 32), 1).astype(jnp.float32)
    texp = jnp.sum((ends_col <= ti).astype(jnp.float32), axis=0, keepdims=True)
    texp_ref[...] = jnp.minimum(texp, E - 1).astype(jnp.int32)[:, :nt]

    pos = pos_ref[...]
    val = TILE * tb_f + pos
    d1 = jnp.sum(jnp.where(elane == s1, val, 0.0), axis=-1, keepdims=True)
    d2 = jnp.sum(jnp.where(elane == s2, val, 0.0), axis=-1, keepdims=True)
    dst_ref[...] = jnp.concatenate([d1, d2], axis=1).astype(jnp.int32)
    wts_ref[...] = jnp.concatenate([w1, w2], axis=1)


def _moe_body(texp_ref, buf_ref, wg_ref, wu_ref, wd_ref, o_ref, acc_ref,
              wgb_ref, wub_ref, wdb_ref, *, nfch, tile):
    j = pl.program_id(0)
    i = pl.program_id(1)

    fresh = (i == 0) | (texp_ref[i] != texp_ref[jnp.maximum(i - 1, 0)])

    @pl.when(fresh)
    def _():
        wgb_ref[...] = wg_ref[0].astype(jnp.bfloat16)
        wub_ref[...] = wu_ref[0].astype(jnp.bfloat16)
        wdb_ref[...] = wd_ref[0].astype(jnp.bfloat16)

    x = buf_ref[...].astype(jnp.bfloat16)
    g = jnp.dot(x, wgb_ref[...], preferred_element_type=jnp.float32)
    u = jnp.dot(x, wub_ref[...], preferred_element_type=jnp.float32)
    pr = (0.5 * g * (1.0 + lax.erf(g * (2.0 ** -0.5)))) * u
    d = jnp.dot(pr.astype(jnp.bfloat16), wdb_ref[...],
                preferred_element_type=jnp.float32)

    @pl.when(j == 0)
    def _():
        acc_ref[pl.ds(i * tile, tile), :] = d.astype(jnp.bfloat16)

    @pl.when((j > 0) & (j < nfch - 1))
    def _():
        acc_ref[pl.ds(i * tile, tile), :] += d.astype(jnp.bfloat16)

    @pl.when(j == nfch - 1)
    def _():
        o_ref[...] = acc_ref[pl.ds(i * tile, tile), :].astype(jnp.float32) + d


def _final_body(xr_ref, y0_ref, y1_ref, w_ref, o_ref):
    w = w_ref[...]
    o_ref[...] = (xr_ref[...] + w[:, 0:1] * y0_ref[...]
                  + w[:, 1:2] * y1_ref[...])


def _sc_dispatch(hs, idx3, CAP):
    N, C = hs.shape
    RPW = N // NW
    mesh = plsc.VectorSubcoreMesh(core_axis_name="c", subcore_axis_name="s")

    @functools.partial(
        pl.kernel,
        out_type=jax.ShapeDtypeStruct((CAP, C), jnp.float32),
        mesh=mesh,
        scratch_types=[
            pltpu.VMEM((TOPK, RPW), jnp.int32),
            pltpu.VMEM((RPW, C), jnp.float32),
            pltpu.SemaphoreType.DMA,
        ],
    )
    def _dispatch(hs_hbm, idx_hbm, buf_hbm, idx_v, rows_v, sem):
        wid = lax.axis_index("s") * 2 + lax.axis_index("c")
        pltpu.sync_copy(idx_hbm.at[wid], idx_v)
        pltpu.sync_copy(hs_hbm.at[pl.ds(wid * RPW, RPW)], rows_v)
        pltpu.async_copy(rows_v, buf_hbm.at[idx_v.at[0]], sem).wait()
        pltpu.async_copy(rows_v, buf_hbm.at[idx_v.at[1]], sem).wait()

    return _dispatch(hs, idx3)


def _sc_combine(obuf, idx3, N):
    C = obuf.shape[1]
    RPW = N // NW
    mesh = plsc.VectorSubcoreMesh(core_axis_name="c", subcore_axis_name="s")

    @functools.partial(
        pl.kernel,
        out_type=[
            jax.ShapeDtypeStruct((N, C), jnp.float32),
            jax.ShapeDtypeStruct((N, C), jnp.float32),
        ],
        mesh=mesh,
        scratch_types=[
            pltpu.VMEM((TOPK, RPW), jnp.int32),
            pltpu.VMEM((RPW, C), jnp.float32),
            pltpu.SemaphoreType.DMA,
        ],
    )
    def _combine(obuf_hbm, idx_hbm, y0_hbm, y1_hbm, idx_v, rows_v, sem):
        wid = lax.axis_index("s") * 2 + lax.axis_index("c")
        pltpu.sync_copy(idx_hbm.at[wid], idx_v)
        pltpu.async_copy(obuf_hbm.at[idx_v.at[0]], rows_v, sem).wait()
        pltpu.sync_copy(rows_v, y0_hbm.at[pl.ds(wid * RPW, RPW)])
        pltpu.async_copy(obuf_hbm.at[idx_v.at[1]], rows_v, sem).wait()
        pltpu.sync_copy(rows_v, y1_hbm.at[pl.ds(wid * RPW, RPW)])

    return _combine(obuf, idx3)


def kernel(x, g1, b1, Wq, Wkv, Wproj, bproj, g2, b2, Wg, feat0, w_gate,
           w_up, w_down, modality_length):
    Bx, N, C = x.shape
    FF = w_gate.shape[2]
    hd = C // H
    scale = hd ** (-0.5)
    CAP = NT * TILE
    RPW = N // NW

    x2 = x.reshape(N, C)
    Wqkv = jnp.concatenate([Wq, Wkv], axis=1)
    g1r, b1r = g1.reshape(1, C), b1.reshape(1, C)
    g2r, b2r = g2.reshape(1, C), b2.reshape(1, C)
    bpr, f0r = bproj.reshape(1, C), feat0.reshape(1, C)

    qkv = pl.pallas_call(
        _ln_qkv_body,
        grid=(N // RT,),
        in_specs=[
            pl.BlockSpec((RT, C), lambda i: (i, 0)),
            pl.BlockSpec((1, C), lambda i: (0, 0)),
            pl.BlockSpec((1, C), lambda i: (0, 0)),
            pl.BlockSpec((C, 3 * C), lambda i: (0, 0)),
        ],
        out_specs=pl.BlockSpec((RT, 3 * C), lambda i: (i, 0)),
        out_shape=jax.ShapeDtypeStruct((N, 3 * C), jnp.float32),
    )(x2, g1r, b1r, Wqkv)

    HP = H // 2
    CB = C // 128
    a = pl.pallas_call(
        functools.partial(_attn_body, scale=scale, hd=hd),
        grid=(HP, N // QT),
        in_specs=[
            pl.BlockSpec((QT, 2 * hd), lambda h, i: (i, h)),
            pl.BlockSpec((N, 2 * hd), lambda h, i: (0, CB + h)),
            pl.BlockSpec((N, 2 * hd), lambda h, i: (0, 2 * CB + h)),
        ],
        out_specs=pl.BlockSpec((QT, 2 * hd), lambda h, i: (i, h)),
        out_shape=jax.ShapeDtypeStruct((N, C), jnp.float32),
    )(qkv, qkv, qkv)

    xres, hs, logits = pl.pallas_call(
        _proj_body,
        grid=(N // RT,),
        in_specs=[
            pl.BlockSpec((RT, C), lambda i: (i, 0)),
            pl.BlockSpec((C, C), lambda i: (0, 0)),
            pl.BlockSpec((1, C), lambda i: (0, 0)),
            pl.BlockSpec((RT, C), lambda i: (i, 0)),
            pl.BlockSpec((1, C), lambda i: (0, 0)),
            pl.BlockSpec((1, C), lambda i: (0, 0)),
            pl.BlockSpec((1, C), lambda i: (0, 0)),
            pl.BlockSpec((C, E), lambda i: (0, 0)),
        ],
        out_specs=[
            pl.BlockSpec((RT, C), lambda i: (i, 0)),
            pl.BlockSpec((RT, C), lambda i: (i, 0)),
            pl.BlockSpec((RT, E), lambda i: (i, 0)),
        ],
        out_shape=[
            jax.ShapeDtypeStruct((N, C), jnp.float32),
            jax.ShapeDtypeStruct((N, C), jnp.float32),
            jax.ShapeDtypeStruct((N, E), jnp.float32),
        ],
    )(a, Wproj, bpr, x2, g2r, b2r, f0r, Wg)

    dst, wts, texp = pl.pallas_call(
        functools.partial(_route_body, T=N, nt=NT),
        in_specs=[pl.BlockSpec((N, E), lambda: (0, 0))],
        out_specs=[
            pl.BlockSpec((N, TOPK), lambda: (0, 0)),
            pl.BlockSpec((N, TOPK), lambda: (0, 0)),
            pl.BlockSpec((1, NT), lambda: (0, 0)),
        ],
        out_shape=[
            jax.ShapeDtypeStruct((N, TOPK), jnp.int32),
            jax.ShapeDtypeStruct((N, TOPK), jnp.float32),
            jax.ShapeDtypeStruct((1, NT), jnp.int32),
        ],
        scratch_shapes=[pltpu.VMEM((N, E), jnp.float32),
                        pltpu.VMEM((N, E), jnp.float32)],
    )(logits)

    texp_flat = texp.reshape(NT)
    idx3 = dst.reshape(NW, RPW, TOPK).transpose(0, 2, 1)

    buf = _sc_dispatch(hs, idx3, CAP)

    nfch = FF // FCH
    grid_spec = pltpu.PrefetchScalarGridSpec(
        num_scalar_prefetch=1,
        grid=(nfch, NT),
        in_specs=[
            pl.BlockSpec((TILE, C), lambda j, i, te: (i, 0)),
            pl.BlockSpec((1, C, FCH), lambda j, i, te: (te[i], 0, j)),
            pl.BlockSpec((1, C, FCH), lambda j, i, te: (te[i], 0, j)),
            pl.BlockSpec((1, FCH, C), lambda j, i, te: (te[i], j, 0)),
        ],
        out_specs=pl.BlockSpec(
            (TILE, C),
            lambda j, i, te: (jnp.where(j == nfch - 1, i, 0), 0)),
        scratch_shapes=[pltpu.VMEM((NT * TILE, C), jnp.bfloat16),
                        pltpu.VMEM((C, FCH), jnp.bfloat16),
                        pltpu.VMEM((C, FCH), jnp.bfloat16),
                        pltpu.VMEM((FCH, C), jnp.bfloat16)],
    )
    obuf = pl.pallas_call(
        functools.partial(_moe_body, nfch=nfch, tile=TILE),
        grid_spec=grid_spec,
        out_shape=jax.ShapeDtypeStruct((CAP, C), jnp.float32),
    )(texp_flat, buf, w_gate, w_up, w_down)

    y0, y1 = _sc_combine(obuf, idx3, N)

    out = pl.pallas_call(
        _final_body,
        grid=(N // RT,),
        in_specs=[
            pl.BlockSpec((RT, C), lambda i: (i, 0)),
            pl.BlockSpec((RT, C), lambda i: (i, 0)),
            pl.BlockSpec((RT, C), lambda i: (i, 0)),
            pl.BlockSpec((RT, TOPK), lambda i: (i, 0)),
        ],
        out_specs=pl.BlockSpec((RT, C), lambda i: (i, 0)),
        out_shape=jax.ShapeDtypeStruct((N, C), jnp.float32),
    )(xres, y0, y1, wts)

    return out.reshape(Bx, N, C)


# speedup vs baseline: 1.4361x; 1.0011x over previous
import functools

import jax
import jax.numpy as jnp
from jax import lax
from jax.experimental import pallas as pl
from jax.experimental.pallas import tpu as pltpu
from jax.experimental.pallas import tpu_sc as plsc

H = 16
E = 8
TOPK = 2
TILE = 256
NT = 24
FCH = 1024
RT = 256
QT = 512
LN_EPS = 1e-5
NW = 32


def _ln(x, g, b):
    mu = jnp.mean(x, axis=-1, keepdims=True)
    var = jnp.mean(jnp.square(x - mu), axis=-1, keepdims=True)
    return (x - mu) * lax.rsqrt(var + LN_EPS) * g + b


def _ln_qkv_body(x_ref, g_ref, b_ref, w_ref, o_ref):
    h = _ln(x_ref[...], g_ref[...], b_ref[...])
    o_ref[...] = jnp.dot(h, w_ref[...], preferred_element_type=jnp.float32)


def _attn_body(q_ref, k_ref, v_ref, o_ref, *, scale, hd):
    outs = []
    for t in range(2):
        q = q_ref[:, t * hd:(t + 1) * hd]
        k = k_ref[:, t * hd:(t + 1) * hd]
        v = v_ref[:, t * hd:(t + 1) * hd]
        s = lax.dot_general(q, k, (((1,), (1,)), ((), ())),
                            preferred_element_type=jnp.float32) * scale
        p = jnp.exp(s)
        r = 1.0 / jnp.sum(p, axis=-1, keepdims=True)
        pv = jnp.dot(p, v, preferred_element_type=jnp.float32)
        outs.append(pv * r)
    o_ref[...] = jnp.concatenate(outs, axis=1)


def _proj_body(a_ref, wp_ref, bp_ref, x_ref, g_ref, b_ref, f0_ref, wg_ref,
               xres_ref, hs_ref, lg_ref):
    xr = x_ref[...] + jnp.dot(a_ref[...], wp_ref[...],
                              preferred_element_type=jnp.float32) + bp_ref[...]
    xres_ref[...] = xr
    hs = _ln(xr, g_ref[...], b_ref[...])
    hs_ref[...] = hs
    lg_ref[...] = jnp.dot(hs + f0_ref[...], wg_ref[...],
                          preferred_element_type=jnp.float32)


def _route_body(lg_ref, dst_ref, wts_ref, texp_ref, pos_ref, oh_ref, *, T, nt):
    lg = lg_ref[...]
    m = jnp.max(lg, axis=-1, keepdims=True)
    p = jnp.exp(lg - m)
    rw = p / jnp.sum(p, axis=-1, keepdims=True)
    elane = lax.broadcasted_iota(jnp.int32, (T, E), 1)
    m1 = jnp.max(rw, axis=-1, keepdims=True)
    s1 = jnp.min(jnp.where(rw == m1, elane, E), axis=-1, keepdims=True)
    rw2 = jnp.where(elane == s1, -1.0, rw)
    m2 = jnp.max(rw2, axis=-1, keepdims=True)
    s2 = jnp.min(jnp.where(rw2 == m2, elane, E), axis=-1, keepdims=True)
    w1 = m1 / (m1 + m2)
    w2 = m2 / (m1 + m2)
    oh_ref[...] = ((elane == s1) | (elane == s2)).astype(jnp.float32)

    CH = 128
    r_i = lax.broadcasted_iota(jnp.int32, (CH, CH), 0)
    c_i = lax.broadcasted_iota(jnp.int32, (CH, CH), 1)
    tril = (r_i > c_i).astype(jnp.float32)

    def chunk_step(c, carry):
        chunk = oh_ref[pl.ds(c * CH, CH), :]
        within = jnp.dot(tril, chunk, preferred_element_type=jnp.float32)
        pos_ref[pl.ds(c * CH, CH), :] = within + carry
        return carry + jnp.sum(chunk, axis=0, keepdims=True)

    counts = lax.fori_loop(0, T // CH, chunk_step,
                           jnp.zeros((1, E), jnp.float32))

    tp = (counts.astype(jnp.int32) + (TILE - 1)) >> 8
    tp_f = tp.astype(jnp.float32)
    a8 = lax.broadcasted_iota(jnp.int32, (E, E), 0)
    b8 = lax.broadcasted_iota(jnp.int32, (E, E), 1)
    mlt = (a8 < b8).astype(jnp.float32)
    tb_f = jnp.dot(tp_f, mlt, preferred_element_type=jnp.float32)
    ends_f = tb_f + tp_f
    eye = (a8 == b8).astype(jnp.float32)
    ends_col = lax.dot_general(eye, ends_f, (((1,), (1,)), ((), ())),
                               preferred_element_type=jnp.float32)
    ti = lax.broadcasted_iota(jnp.int32, (E, 32), 1).astype(jnp.float32)
    texp = jnp.sum((ends_col <= ti).astype(jnp.float32), axis=0, keepdims=True)
    texp_ref[...] = jnp.minimum(texp, E - 1).astype(jnp.int32)[:, :nt]

    pos = pos_ref[...]
    val = TILE * tb_f + pos
    d1 = jnp.sum(jnp.where(elane == s1, val, 0.0), axis=-1, keepdims=True)
    d2 = jnp.sum(jnp.where(elane == s2, val, 0.0), axis=-1, keepdims=True)
    dst_ref[...] = jnp.concatenate([d1, d2], axis=1).astype(jnp.int32)
    wts_ref[...] = jnp.concatenate([w1, w2], axis=1)


def _moe_body(texp_ref, buf_ref, wg_ref, wu_ref, wd_ref, o_ref, acc_ref,
              wgb_ref, wub_ref, wdb_ref, *, nfch, tile):
    j = pl.program_id(0)
    i = pl.program_id(1)

    fresh = (i == 0) | (texp_ref[i] != texp_ref[jnp.maximum(i - 1, 0)])

    @pl.when(fresh)
    def _():
        wgb_ref[...] = wg_ref[0].astype(jnp.bfloat16)
        wub_ref[...] = wu_ref[0].astype(jnp.bfloat16)
        wdb_ref[...] = wd_ref[0].astype(jnp.bfloat16)

    x = buf_ref[...].astype(jnp.bfloat16)
    g = jnp.dot(x, wgb_ref[...], preferred_element_type=jnp.float32)
    u = jnp.dot(x, wub_ref[...], preferred_element_type=jnp.float32)
    pr = (0.5 * g * (1.0 + lax.erf(g * (2.0 ** -0.5)))) * u
    d = jnp.dot(pr.astype(jnp.bfloat16), wdb_ref[...],
                preferred_element_type=jnp.float32)

    @pl.when(j == 0)
    def _():
        acc_ref[pl.ds(i * tile, tile), :] = d.astype(jnp.bfloat16)

    @pl.when((j > 0) & (j < nfch - 1))
    def _():
        acc_ref[pl.ds(i * tile, tile), :] += d.astype(jnp.bfloat16)

    @pl.when(j == nfch - 1)
    def _():
        o_ref[...] = acc_ref[pl.ds(i * tile, tile), :].astype(jnp.float32) + d


def _final_body(xr_ref, y0_ref, y1_ref, w_ref, o_ref):
    w = w_ref[...]
    o_ref[...] = (xr_ref[...] + w[:, 0:1] * y0_ref[...]
                  + w[:, 1:2] * y1_ref[...])


def _sc_dispatch(hs, idx3, CAP):
    N, C = hs.shape
    RPW = N // NW
    mesh = plsc.VectorSubcoreMesh(core_axis_name="c", subcore_axis_name="s")

    @functools.partial(
        pl.kernel,
        out_type=jax.ShapeDtypeStruct((CAP, C), jnp.float32),
        mesh=mesh,
        scratch_types=[
            pltpu.VMEM((TOPK, RPW), jnp.int32),
            pltpu.VMEM((RPW, C), jnp.float32),
            pltpu.SemaphoreType.DMA,
        ],
    )
    def _dispatch(hs_hbm, idx_hbm, buf_hbm, idx_v, rows_v, sem):
        wid = lax.axis_index("s") * 2 + lax.axis_index("c")
        pltpu.sync_copy(idx_hbm.at[wid], idx_v)
        pltpu.sync_copy(hs_hbm.at[pl.ds(wid * RPW, RPW)], rows_v)
        pltpu.async_copy(rows_v, buf_hbm.at[idx_v.at[0]], sem).wait()
        pltpu.async_copy(rows_v, buf_hbm.at[idx_v.at[1]], sem).wait()

    return _dispatch(hs, idx3)


def _sc_combine(obuf, idx3, N):
    C = obuf.shape[1]
    RPW = N // NW
    mesh = plsc.VectorSubcoreMesh(core_axis_name="c", subcore_axis_name="s")

    @functools.partial(
        pl.kernel,
        out_type=[
            jax.ShapeDtypeStruct((N, C), jnp.float32),
            jax.ShapeDtypeStruct((N, C), jnp.float32),
        ],
        mesh=mesh,
        scratch_types=[
            pltpu.VMEM((TOPK, RPW), jnp.int32),
            pltpu.VMEM((RPW, C), jnp.float32),
            pltpu.SemaphoreType.DMA,
        ],
    )
    def _combine(obuf_hbm, idx_hbm, y0_hbm, y1_hbm, idx_v, rows_v, sem):
        wid = lax.axis_index("s") * 2 + lax.axis_index("c")
        pltpu.sync_copy(idx_hbm.at[wid], idx_v)
        pltpu.async_copy(obuf_hbm.at[idx_v.at[0]], rows_v, sem).wait()
        pltpu.sync_copy(rows_v, y0_hbm.at[pl.ds(wid * RPW, RPW)])
        pltpu.async_copy(obuf_hbm.at[idx_v.at[1]], rows_v, sem).wait()
        pltpu.sync_copy(rows_v, y1_hbm.at[pl.ds(wid * RPW, RPW)])

    return _combine(obuf, idx3)


def kernel(x, g1, b1, Wq, Wkv, Wproj, bproj, g2, b2, Wg, feat0, w_gate,
           w_up, w_down, modality_length):
    Bx, N, C = x.shape
    FF = w_gate.shape[2]
    hd = C // H
    scale = hd ** (-0.5)
    CAP = NT * TILE
    RPW = N // NW

    x2 = x.reshape(N, C)
    Wqkv = jnp.concatenate([Wq, Wkv], axis=1)
    g1r, b1r = g1.reshape(1, C), b1.reshape(1, C)
    g2r, b2r = g2.reshape(1, C), b2.reshape(1, C)
    bpr, f0r = bproj.reshape(1, C), feat0.reshape(1, C)

    qkv = pl.pallas_call(
        _ln_qkv_body,
        grid=(N // RT,),
        in_specs=[
            pl.BlockSpec((RT, C), lambda i: (i, 0)),
            pl.BlockSpec((1, C), lambda i: (0, 0)),
            pl.BlockSpec((1, C), lambda i: (0, 0)),
            pl.BlockSpec((C, 3 * C), lambda i: (0, 0)),
        ],
        out_specs=pl.BlockSpec((RT, 3 * C), lambda i: (i, 0)),
        out_shape=jax.ShapeDtypeStruct((N, 3 * C), jnp.float32),
    )(x2, g1r, b1r, Wqkv)

    HP = H // 2
    CB = C // 128
    a = pl.pallas_call(
        functools.partial(_attn_body, scale=scale, hd=hd),
        grid=(HP, N // QT),
        in_specs=[
            pl.BlockSpec((QT, 2 * hd), lambda h, i: (i, h)),
            pl.BlockSpec((N, 2 * hd), lambda h, i: (0, CB + h)),
            pl.BlockSpec((N, 2 * hd), lambda h, i: (0, 2 * CB + h)),
        ],
        out_specs=pl.BlockSpec((QT, 2 * hd), lambda h, i: (i, h)),
        out_shape=jax.ShapeDtypeStruct((N, C), jnp.float32),
    )(qkv, qkv, qkv)

    xres, hs, logits = pl.pallas_call(
        _proj_body,
        grid=(N // RT,),
        in_specs=[
            pl.BlockSpec((RT, C), lambda i: (i, 0)),
            pl.BlockSpec((C, C), lambda i: (0, 0)),
            pl.BlockSpec((1, C), lambda i: (0, 0)),
            pl.BlockSpec((RT, C), lambda i: (i, 0)),
            pl.BlockSpec((1, C), lambda i: (0, 0)),
            pl.BlockSpec((1, C), lambda i: (0, 0)),
            pl.BlockSpec((1, C), lambda i: (0, 0)),
            pl.BlockSpec((C, E), lambda i: (0, 0)),
        ],
        out_specs=[
            pl.BlockSpec((RT, C), lambda i: (i, 0)),
            pl.BlockSpec((RT, C), lambda i: (i, 0)),
            pl.BlockSpec((RT, E), lambda i: (i, 0)),
        ],
        out_shape=[
            jax.ShapeDtypeStruct((N, C), jnp.float32),
            jax.ShapeDtypeStruct((N, C), jnp.float32),
            jax.ShapeDtypeStruct((N, E), jnp.float32),
        ],
    )(a, Wproj, bpr, x2, g2r, b2r, f0r, Wg)

    dst, wts, texp = pl.pallas_call(
        functools.partial(_route_body, T=N, nt=NT),
        in_specs=[pl.BlockSpec((N, E), lambda: (0, 0))],
        out_specs=[
            pl.BlockSpec((N, TOPK), lambda: (0, 0)),
            pl.BlockSpec((N, TOPK), lambda: (0, 0)),
            pl.BlockSpec((1, NT), lambda: (0, 0)),
        ],
        out_shape=[
            jax.ShapeDtypeStruct((N, TOPK), jnp.int32),
            jax.ShapeDtypeStruct((N, TOPK), jnp.float32),
            jax.ShapeDtypeStruct((1, NT), jnp.int32),
        ],
        scratch_shapes=[pltpu.VMEM((N, E), jnp.float32),
                        pltpu.VMEM((N, E), jnp.float32)],
    )(logits)

    texp_flat = texp.reshape(NT)
    idx3 = dst.reshape(NW, RPW, TOPK).transpose(0, 2, 1)

    buf = _sc_dispatch(hs, idx3, CAP)

    nfch = FF // FCH
    grid_spec = pltpu.PrefetchScalarGridSpec(
        num_scalar_prefetch=1,
        grid=(nfch, NT),
        in_specs=[
            pl.BlockSpec((TILE, C), lambda j, i, te: (i, 0)),
            pl.BlockSpec((1, C, FCH), lambda j, i, te: (te[i], 0, j)),
            pl.BlockSpec((1, C, FCH), lambda j, i, te: (te[i], 0, j)),
            pl.BlockSpec((1, FCH, C), lambda j, i, te: (te[i], j, 0)),
        ],
        out_specs=pl.BlockSpec(
            (TILE, C),
            lambda j, i, te: (jnp.where(j == nfch - 1, i, 0), 0)),
        scratch_shapes=[pltpu.VMEM((NT * TILE, C), jnp.bfloat16),
                        pltpu.VMEM((C, FCH), jnp.bfloat16),
                        pltpu.VMEM((C, FCH), jnp.bfloat16),
                        pltpu.VMEM((FCH, C), jnp.bfloat16)],
    )
    obuf = pl.pallas_call(
        functools.partial(_moe_body, nfch=nfch, tile=TILE),
        grid_spec=grid_spec,
        out_shape=jax.ShapeDtypeStruct((CAP, C), jnp.float32),
    )(texp_flat, buf, w_gate, w_up, w_down)

    y0, y1 = _sc_combine(obuf, idx3, N)

    out = pl.pallas_call(
        _final_body,
        grid=(N // RT,),
        in_specs=[
            pl.BlockSpec((RT, C), lambda i: (i, 0)),
            pl.BlockSpec((RT, C), lambda i: (i, 0)),
            pl.BlockSpec((RT, C), lambda i: (i, 0)),
            pl.BlockSpec((RT, TOPK), lambda i: (i, 0)),
        ],
        out_specs=pl.BlockSpec((RT, C), lambda i: (i, 0)),
        out_shape=jax.ShapeDtypeStruct((N, C), jnp.float32),
    )(xres, y0, y1, wts)

    return out.reshape(Bx, N, C)
